# baseline (speedup 1.0000x reference)
import functools

import jax
import jax.numpy as jnp
from jax import lax
from jax.experimental import pallas as pl
from jax.experimental.pallas import tpu as pltpu
from jax.experimental.pallas import tpu_sc as plsc

_VOCAB = 1000000
_D = 64
_B = 1024
_S = 200
_NBB = 8
_BBLK = _B // _NBB
_SGO = (0, 48, 96, 144)
_SGN = 56
_L = 16


def _emb_body(textT, tab2, posT, outT, idxs, idx2, pbuf, stg, pos_v, pos_rm,
              gsem0, gsem1, ssem0, ssem1):
    wid = lax.axis_index("s") * 2 + lax.axis_index("c")
    bb = lax.rem(wid, _NBB)
    sq = wid // _NBB
    s_off = sq * 48
    gsem = (gsem0, gsem1)
    ssem = (ssem0, ssem1)

    pltpu.sync_copy(
        textT.at[pl.ds(s_off, _SGN), pl.ds(bb * _BBLK, _BBLK)], idxs)
    pltpu.sync_copy(posT, pos_v)

    iota64 = lax.iota(jnp.int32, _L) * _D

    _POS_OFF = tuple(range(0, _S - _L, _L)) + (_S - _L,)

    @plsc.parallel_loop(0, _D)
    def _pos_t(c):
        for off in _POS_OFF:
            v = pos_v[c, pl.ds(off, _L)]
            plsc.store_scatter(
                pos_rm,
                [lax.iota(jnp.int32, _L) + off,
                 jnp.full((_L,), c, jnp.int32)],
                v,
            )

    iota16 = lax.iota(jnp.int32, _L)

    def prep_and_fire(si, buf):
        for g in range(_BBLK // _L):
            ids = idxs[si, pl.ds(g * _L, _L)]
            idx2[buf, pl.ds(g * _L, _L)] = lax.shift_right_logical(ids, 1)
        pltpu.async_copy(tab2.at[idx2.at[buf]], pbuf.at[buf], gsem[buf])

    def wait_gather(buf):
        pltpu.make_async_copy(
            tab2.at[pl.ds(0, _BBLK)], pbuf.at[buf], gsem[buf]).wait()

    def wait_store(buf):
        pltpu.make_async_copy(
            stg.at[buf], outT.at[0, :, pl.ds(0, _BBLK)], ssem[buf]).wait()

    def process(si, buf, sbuf):
        s = s_off + si
        pvq = [pos_rm[s, pl.ds(q * _L, _L)] for q in range(_D // _L)]
        hcol = []
        rowv = []
        for g in range(_BBLK // _L):
            ids = idxs[si, pl.ds(g * _L, _L)]
            hcol.append(
                lax.mul(lax.rem(ids, 2), jnp.int32(_D)))
            rowv.append(iota16 + g * _L)
        for c in range(_D):
            pspl = jnp.full((_L,), pvq[c // _L][c % _L], jnp.float32)
            for g in range(_BBLK // _L):
                col = hcol[g] + c
                v = plsc.load_gather(pbuf.at[buf], [rowv[g], col])
                stg[sbuf, c, pl.ds(g * _L, _L)] = v + pspl

    def store(si, sbuf):
        pltpu.async_copy(
            stg.at[sbuf],
            outT.at[s_off + si, :, pl.ds(bb * _BBLK, _BBLK)],
            ssem[sbuf],
        )

    prep_and_fire(0, 0)

    def body(gi, carry):
        for b in range(2):
            si = gi * 2 + b

            @pl.when(si + 1 < _SGN)
            def _():
                prep_and_fire(si + 1, 1 - b)

            wait_gather(b)

            @pl.when(si >= 2)
            def _():
                wait_store(b)

            process(si, b, b)
            store(si, b)
        return carry

    lax.fori_loop(0, _SGN // 2, body, 0)
    wait_store(0)
    wait_store(1)


@jax.jit
def _emb(textT, tab2, posT):
    mesh = plsc.VectorSubcoreMesh(core_axis_name="c", subcore_axis_name="s")
    f = functools.partial(
        pl.kernel,
        mesh=mesh,
        out_type=jax.ShapeDtypeStruct((_S, _D, _B), jnp.float32),
        scratch_types=[
            pltpu.VMEM((_SGN, _BBLK), jnp.int32),
            pltpu.VMEM((2, _BBLK), jnp.int32),
            pltpu.VMEM((2, _BBLK, 2 * _D), jnp.float32),
            pltpu.VMEM((2, _D, _BBLK), jnp.float32),
            pltpu.VMEM((_D, _S), jnp.float32),
            pltpu.VMEM((_S, _D), jnp.float32),
            pltpu.SemaphoreType.DMA,
            pltpu.SemaphoreType.DMA,
            pltpu.SemaphoreType.DMA,
            pltpu.SemaphoreType.DMA,
        ],
        compiler_params=pltpu.CompilerParams(
            use_tc_tiling_on_sc=True, needs_layout_passes=False
        ),
    )(_emb_body)
    return f(textT, tab2, posT)


def kernel(text, token_table, pos_embedding):
    textT = text.T
    tab2 = token_table.reshape(_VOCAB // 2, 2 * _D)
    posT = pos_embedding.T
    outT = _emb(textT, tab2, posT)
    return outT.transpose(2, 0, 1)


# speedup vs baseline: 1.0924x; 1.0924x over previous
import functools

import jax
import jax.numpy as jnp
from jax import lax
from jax.experimental import pallas as pl
from jax.experimental.pallas import tpu as pltpu
from jax.experimental.pallas import tpu_sc as plsc

_VOCAB = 1000000
_D = 64
_B = 1024
_S = 200
_NBB = 8
_BBLK = _B // _NBB
_SGO = (0, 48, 96, 144)
_SGN = 56
_L = 16


def _emb_body(textT, tab2, posT, outT, idxs, idx2, pbuf, stg, pos_v, pos_rm,
              gsem0, gsem1, ssem0, ssem1):
    wid = lax.axis_index("s") * 2 + lax.axis_index("c")
    bb = lax.rem(wid, _NBB)
    sq = wid // _NBB
    s_off = sq * 48
    gsem = (gsem0, gsem1)
    ssem = (ssem0, ssem1)

    pltpu.sync_copy(
        textT.at[pl.ds(s_off, _SGN), pl.ds(bb * _BBLK, _BBLK)], idxs)
    pltpu.sync_copy(posT, pos_v)

    iota64 = lax.iota(jnp.int32, _L) * _D

    _POS_OFF = tuple(range(0, _S - _L, _L)) + (_S - _L,)

    @plsc.parallel_loop(0, _D)
    def _pos_t(c):
        for off in _POS_OFF:
            v = pos_v[c, pl.ds(off, _L)]
            plsc.store_scatter(
                pos_rm,
                [lax.iota(jnp.int32, _L) + off,
                 jnp.full((_L,), c, jnp.int32)],
                v,
            )

    iota16 = lax.iota(jnp.int32, _L)

    def prep_and_fire(si, buf):
        for g in range(_BBLK // _L):
            ids = idxs[si, pl.ds(g * _L, _L)]
            idx2[buf, pl.ds(g * _L, _L)] = lax.shift_right_logical(ids, 1)
        pltpu.async_copy(tab2.at[idx2.at[buf]], pbuf.at[buf], gsem[buf])

    def wait_gather(buf):
        pltpu.make_async_copy(
            tab2.at[pl.ds(0, _BBLK)], pbuf.at[buf], gsem[buf]).wait()

    def wait_store(buf):
        pltpu.make_async_copy(
            stg.at[buf], outT.at[0, :, pl.ds(0, _BBLK)], ssem[buf]).wait()

    def process(si, buf, sbuf):
        s = s_off + si
        hcol = []
        rowv = []
        for g in range(_BBLK // _L):
            ids = idxs[si, pl.ds(g * _L, _L)]
            hcol.append(
                lax.mul(lax.rem(ids, 2), jnp.int32(_D)))
            rowv.append(iota16 + g * _L)
        @plsc.parallel_loop(0, _D // _L)
        def _cols(q):
            pq = pos_rm[s, pl.ds(q * _L, _L)]
            for cl in range(_L):
                c = q * _L + cl
                pspl = jnp.full((_L,), pq[cl], jnp.float32)
                for g in range(_BBLK // _L):
                    col = hcol[g] + c
                    v = plsc.load_gather(pbuf.at[buf], [rowv[g], col])
                    stg[sbuf, c, pl.ds(g * _L, _L)] = v + pspl

    def store(si, sbuf):
        pltpu.async_copy(
            stg.at[sbuf],
            outT.at[s_off + si, :, pl.ds(bb * _BBLK, _BBLK)],
            ssem[sbuf],
        )

    prep_and_fire(0, 0)

    def body(gi, carry):
        for b in range(2):
            si = gi * 2 + b

            @pl.when(si + 1 < _SGN)
            def _():
                prep_and_fire(si + 1, 1 - b)

            wait_gather(b)

            @pl.when(si >= 2)
            def _():
                wait_store(b)

            process(si, b, b)
            store(si, b)
        return carry

    lax.fori_loop(0, _SGN // 2, body, 0)
    wait_store(0)
    wait_store(1)


@jax.jit
def _emb(textT, tab2, posT):
    mesh = plsc.VectorSubcoreMesh(core_axis_name="c", subcore_axis_name="s")
    f = functools.partial(
        pl.kernel,
        mesh=mesh,
        out_type=jax.ShapeDtypeStruct((_S, _D, _B), jnp.float32),
        scratch_types=[
            pltpu.VMEM((_SGN, _BBLK), jnp.int32),
            pltpu.VMEM((2, _BBLK), jnp.int32),
            pltpu.VMEM((2, _BBLK, 2 * _D), jnp.float32),
            pltpu.VMEM((2, _D, _BBLK), jnp.float32),
            pltpu.VMEM((_D, _S), jnp.float32),
            pltpu.VMEM((_S, _D), jnp.float32),
            pltpu.SemaphoreType.DMA,
            pltpu.SemaphoreType.DMA,
            pltpu.SemaphoreType.DMA,
            pltpu.SemaphoreType.DMA,
        ],
        compiler_params=pltpu.CompilerParams(
            use_tc_tiling_on_sc=True, needs_layout_passes=False
        ),
    )(_emb_body)
    return f(textT, tab2, posT)


def kernel(text, token_table, pos_embedding):
    textT = text.T
    tab2 = token_table.reshape(_VOCAB // 2, 2 * _D)
    posT = pos_embedding.T
    outT = _emb(textT, tab2, posT)
    return outT.transpose(2, 0, 1)


# speedup vs baseline: 1.2953x; 1.1858x over previous
import functools

import jax
import jax.numpy as jnp
from jax import lax
from jax.experimental import pallas as pl
from jax.experimental.pallas import tpu as pltpu
from jax.experimental.pallas import tpu_sc as plsc

_VOCAB = 1000000
_D = 64
_B = 1024
_S = 200
_NC = 2
_NS = 16
_NW = _NC * _NS
_ROWS_PER_W = _B // _NW
_CHUNKS = ((0, 104), (104, 96))
_LANES = 16
_G = 4
_NSTEP = _ROWS_PER_W // _G
_NBUF = 2


def _emb_body(text_hbm, table_hbm, pos_hbm, out_hbm,
              idx_all, rows_buf, pos_v, gsem0, gsem1, ssem0, ssem1):
    wid = lax.axis_index("s") * _NC + lax.axis_index("c")
    row0 = wid * _ROWS_PER_W
    gsem = (gsem0, gsem1)
    ssem = (ssem0, ssem1)

    pltpu.sync_copy(text_hbm.at[pl.ds(row0, _ROWS_PER_W)], idx_all)
    pltpu.sync_copy(pos_hbm, pos_v)

    def issue_gathers(step, buf):
        for j in range(_G):
            for off, n in _CHUNKS:
                pltpu.async_copy(
                    table_hbm.at[idx_all.at[step * _G + j, pl.ds(off, n)]],
                    rows_buf.at[buf, j, pl.ds(off, n)],
                    gsem[buf],
                )

    def wait_gathers(buf):
        for j in range(_G):
            pltpu.make_async_copy(
                table_hbm.at[pl.ds(0, _S)], rows_buf.at[buf, j], gsem[buf]
            ).wait()

    def issue_store(step, buf):
        pltpu.async_copy(
            rows_buf.at[buf],
            out_hbm.at[pl.ds(row0 + step * _G, _G)],
            ssem[buf],
        )

    def wait_store(buf):
        pltpu.make_async_copy(
            rows_buf.at[buf], out_hbm.at[pl.ds(0, _G)], ssem[buf]
        ).wait()

    def add_pos(buf):
        @plsc.parallel_loop(0, _S, unroll=2)
        def body_r(r):
            for q in range(_D // _LANES):
                cols = pl.ds(q * _LANES, _LANES)
                p = pos_v[r, cols]
                for j in range(_G):
                    sl = (buf, j, r, cols)
                    rows_buf[sl] = rows_buf[sl] + p

    def slot(b, step):
        nb = 1 - b

        @pl.when(step + 1 < _NSTEP)
        def _():
            @pl.when(step >= 1)
            def _():
                wait_store(nb)

            issue_gathers(step + 1, nb)

        wait_gathers(b)
        add_pos(b)
        issue_store(step, b)

    issue_gathers(0, 0)

    def body(gi, carry):
        slot(0, gi * _NBUF)
        slot(1, gi * _NBUF + 1)
        return carry

    lax.fori_loop(0, _NSTEP // _NBUF, body, 0)
    wait_store(0)
    wait_store(1)


@jax.jit
def _emb(text2, table, pos):
    mesh = plsc.VectorSubcoreMesh(core_axis_name="c", subcore_axis_name="s")
    f = functools.partial(
        pl.kernel,
        mesh=mesh,
        out_type=jax.ShapeDtypeStruct((_B, _S, _D), jnp.float32),
        scratch_types=[
            pltpu.VMEM((_ROWS_PER_W, _S), jnp.int32),
            pltpu.VMEM((_NBUF, _G, _S, _D), jnp.float32),
            pltpu.VMEM((_S, _D), jnp.float32),
            pltpu.SemaphoreType.DMA,
            pltpu.SemaphoreType.DMA,
            pltpu.SemaphoreType.DMA,
            pltpu.SemaphoreType.DMA,
        ],
        compiler_params=pltpu.CompilerParams(
            use_tc_tiling_on_sc=False, skip_device_barrier=True
        ),
    )(_emb_body)
    return f(text2, table, pos)


def kernel(text, token_table, pos_embedding):
    return _emb(text, token_table, pos_embedding)
